# Initial kernel scaffold; baseline (speedup 1.0000x reference)
import jax
import jax.numpy as jnp
from jax.experimental import pallas as pl


def kernel(prev_outputs, prev_is_active, parent_indices, w, b):
    raise NotImplementedError("write your pallas kernel here")


# speedup vs baseline: 13.2212x; 13.2212x over previous
import functools

import jax
import jax.numpy as jnp
from jax import lax
from jax.experimental import pallas as pl
from jax.experimental.pallas import tpu as pltpu
from jax.experimental.pallas import tpu_sc as plsc

K = 8192
M = 8192
C = 16
N = 16
NN = N * N
THRESH = 12

NUM_CORES = 2
NUM_SUBCORES = 16
NW = NUM_CORES * NUM_SUBCORES
KPW = K // NW


def _decoder_body(prev_hbm, flags_hbm, idx_hbm, w_hbm, b_hbm,
                  out_hbm, act_hbm,
                  idx_v, w_v, b_v, flags_v, buf_v, ob_v, act_v,
                  sem):
    cid = lax.axis_index("c")
    sid = lax.axis_index("s")
    wid = sid * NUM_CORES + cid
    base = wid * KPW

    pltpu.sync_copy(idx_hbm.at[pl.ds(base, KPW)], idx_v)
    pltpu.sync_copy(w_hbm.at[pl.ds(base, KPW)], w_v)
    pltpu.sync_copy(b_hbm.at[pl.ds(base, KPW)], b_v.at[pl.ds(0, KPW)])
    pltpu.sync_copy(flags_hbm, flags_v)

    def body(k, carry):
        idxv = idx_v[k]
        pltpu.async_copy(prev_hbm.at[idxv], buf_v, sem).wait()
        fl = plsc.load_gather(flags_v, [idxv])
        nact = jnp.sum(fl)
        g = jnp.where(nact >= THRESH - 0.5, 1.0, 0.0).astype(jnp.float32)
        wf = w_v[k] * fl * g
        bk = b_v[pl.ds(k, C)][0] * g
        lane0 = lax.iota(jnp.int32, C) == 0
        plsc.store_scatter(act_v, [jnp.full((C,), k, jnp.int32)],
                           jnp.full((C,), g, jnp.float32), mask=lane0)
        for j in range(N):
            acc = jnp.zeros((N,), jnp.float32) + bk
            for c in range(C):
                acc = acc + wf[c] * buf_v[c, pl.ds(j * N, N)]
            a = jnp.abs(acc)
            e = jnp.exp(-2.0 * a)
            t = (1.0 - e) / (1.0 + e)
            ob_v[pl.ds(j * N, N)] = jnp.sign(acc) * t
        pltpu.sync_copy(ob_v, out_hbm.at[base + k])
        return carry

    lax.fori_loop(0, KPW, body, 0)

    pltpu.sync_copy(act_v, act_hbm.at[pl.ds(base, KPW)])


@jax.jit
def _decoder_layer_sc(prev_flat, flags_f32, parent_indices, w, b):
    mesh = plsc.VectorSubcoreMesh(core_axis_name="c", subcore_axis_name="s")
    return pl.kernel(
        _decoder_body,
        out_type=(
            jax.ShapeDtypeStruct((K, NN), jnp.float32),
            jax.ShapeDtypeStruct((K,), jnp.float32),
        ),
        mesh=mesh,
        compiler_params=pltpu.CompilerParams(needs_layout_passes=False),
        scratch_types=[
            pltpu.VMEM((KPW, C), jnp.int32),
            pltpu.VMEM((KPW, C), jnp.float32),
            pltpu.VMEM((KPW + C,), jnp.float32),
            pltpu.VMEM((M,), jnp.float32),
            pltpu.VMEM((C, NN), jnp.float32),
            pltpu.VMEM((NN,), jnp.float32),
            pltpu.VMEM((KPW,), jnp.float32),
            pltpu.SemaphoreType.DMA,
        ],
    )(prev_flat, flags_f32, parent_indices, w, b)


def kernel(prev_outputs, prev_is_active, parent_indices, w, b):
    prev_flat = prev_outputs.reshape(M, NN)
    flags_f32 = prev_is_active.astype(jnp.float32)
    out_flat, act = _decoder_layer_sc(prev_flat, flags_f32,
                                      parent_indices, w, b)
    return out_flat.reshape(K, N, N), act > 0.5


# speedup vs baseline: 21.3372x; 1.6139x over previous
import functools

import jax
import jax.numpy as jnp
from jax import lax
from jax.experimental import pallas as pl
from jax.experimental.pallas import tpu as pltpu
from jax.experimental.pallas import tpu_sc as plsc

K = 8192
M = 8192
C = 16
N = 16
NN = N * N
THRESH = 12

NUM_CORES = 2
NUM_SUBCORES = 16
NW = NUM_CORES * NUM_SUBCORES
KPW = K // NW
G = 4
NB = KPW // G


def _decoder_body(prev_hbm, flags_hbm, idx_hbm, w_hbm, b_hbm,
                  out_hbm, act_hbm,
                  idx_v, w_v, b_v, flags_v, buf0, buf1, ob0, ob1, act_v,
                  semg0, semg1, semo0, semo1):
    cid = lax.axis_index("c")
    sid = lax.axis_index("s")
    wid = sid * NUM_CORES + cid
    base = wid * KPW

    pltpu.sync_copy(idx_hbm.at[pl.ds(base * C, KPW * C)], idx_v)
    pltpu.sync_copy(w_hbm.at[pl.ds(base * C, KPW * C)], w_v)
    pltpu.sync_copy(b_hbm.at[pl.ds(base, KPW)], b_v.at[pl.ds(0, KPW)])
    pltpu.sync_copy(flags_hbm, flags_v)

    def start_gather(b, buf, sem):
        pltpu.async_copy(prev_hbm.at[idx_v.at[pl.ds(b * (G * C), G * C)]],
                         buf, sem)

    def wait_gather(buf, sem):
        pltpu.make_async_copy(prev_hbm.at[pl.ds(0, G * C)], buf, sem).wait()

    def start_out(ob, b, sem):
        pltpu.async_copy(ob, out_hbm.at[pl.ds(base + b * G, G)], sem)

    def wait_out(ob, sem):
        pltpu.make_async_copy(ob, out_hbm.at[pl.ds(base, G)], sem).wait()

    def compute_batch(b, buf, ob):
        for d in range(G):
            k = b * G + d
            idxv = idx_v[pl.ds(k * C, C)]
            fl = plsc.load_gather(flags_v, [idxv])
            nact = jnp.sum(fl)
            g = jnp.where(nact >= THRESH - 0.5, 1.0, 0.0)
            g = g.astype(jnp.float32)
            wf = w_v[pl.ds(k * C, C)] * fl * g
            bk = b_v[pl.ds(k, C)][0] * g
            lane0 = lax.iota(jnp.int32, C) == 0
            plsc.store_scatter(act_v, [jnp.full((C,), k, jnp.int32)],
                               jnp.full((C,), g, jnp.float32), mask=lane0)
            for j in range(N):
                acc = jnp.zeros((N,), jnp.float32) + bk
                for c in range(C):
                    acc = acc + wf[c] * buf[d * C + c, pl.ds(j * N, N)]
                a = jnp.abs(acc)
                e = jnp.exp(-2.0 * a)
                t = (1.0 - e) / (1.0 + e)
                ob[d, pl.ds(j * N, N)] = jnp.sign(acc) * t

    start_gather(0, buf0, semg0)
    start_gather(1, buf1, semg1)
    start_out(ob0, NB - 2, semo0)
    start_out(ob1, NB - 1, semo1)

    def body(i, carry):
        b0 = 2 * i
        b1 = 2 * i + 1
        wait_gather(buf0, semg0)
        wait_out(ob0, semo0)
        compute_batch(b0, buf0, ob0)
        start_gather(jnp.minimum(b0 + 2, NB - 1), buf0, semg0)
        start_out(ob0, b0, semo0)
        wait_gather(buf1, semg1)
        wait_out(ob1, semo1)
        compute_batch(b1, buf1, ob1)
        start_gather(jnp.minimum(b1 + 2, NB - 1), buf1, semg1)
        start_out(ob1, b1, semo1)
        return carry

    lax.fori_loop(0, NB // 2, body, 0)

    wait_gather(buf0, semg0)
    wait_gather(buf1, semg1)
    wait_out(ob0, semo0)
    wait_out(ob1, semo1)

    pltpu.sync_copy(act_v, act_hbm.at[pl.ds(base, KPW)])


@jax.jit
def _decoder_layer_sc(prev_flat, flags_f32, idx_flat, w_flat, b):
    mesh = plsc.VectorSubcoreMesh(core_axis_name="c", subcore_axis_name="s")
    return pl.kernel(
        _decoder_body,
        out_type=(
            jax.ShapeDtypeStruct((K, NN), jnp.float32),
            jax.ShapeDtypeStruct((K,), jnp.float32),
        ),
        mesh=mesh,
        compiler_params=pltpu.CompilerParams(needs_layout_passes=False),
        scratch_types=[
            pltpu.VMEM((KPW * C,), jnp.int32),
            pltpu.VMEM((KPW * C,), jnp.float32),
            pltpu.VMEM((KPW + C,), jnp.float32),
            pltpu.VMEM((M,), jnp.float32),
            pltpu.VMEM((G * C, NN), jnp.float32),
            pltpu.VMEM((G * C, NN), jnp.float32),
            pltpu.VMEM((G, NN), jnp.float32),
            pltpu.VMEM((G, NN), jnp.float32),
            pltpu.VMEM((KPW,), jnp.float32),
            pltpu.SemaphoreType.DMA,
            pltpu.SemaphoreType.DMA,
            pltpu.SemaphoreType.DMA,
            pltpu.SemaphoreType.DMA,
        ],
    )(prev_flat, flags_f32, idx_flat, w_flat, b)


def kernel(prev_outputs, prev_is_active, parent_indices, w, b):
    prev_flat = prev_outputs.reshape(M, NN)
    flags_f32 = prev_is_active.astype(jnp.float32)
    idx_flat = parent_indices.reshape(K * C)
    w_flat = w.reshape(K * C)
    out_flat, act = _decoder_layer_sc(prev_flat, flags_f32, idx_flat,
                                      w_flat, b)
    return out_flat.reshape(K, N, N), act > 0.5
